# baseline (speedup 1.0000x reference)
import functools

import jax
import jax.numpy as jnp
from jax import lax
from jax.experimental import pallas as pl
from jax.experimental.pallas import tpu as pltpu
from jax.experimental.pallas import tpu_sc as plsc

_N = 10000
_N_PAD = 10240
_E = 320000
_E_PER_W = _E // 32
_SEQ = 500
_SEQ_P = 512
_D = 128
_NW = 32
_L = 16
_NR = 20
_RR = 512
_G = 32

_MESH = plsc.VectorSubcoreMesh(core_axis_name="c", subcore_axis_name="s")
_PARAMS = pltpu.CompilerParams(needs_layout_passes=False,
                               use_tc_tiling_on_sc=False)
_PK = plsc.PackFormat.INTERLEAVED


def _wid():
  return lax.axis_index("c") * 16 + lax.axis_index("s")


def _a_body(edge_hbm, pb_hbm, idx_v, flags_v, pb_v, sem):
  w = _wid()
  zeros = jnp.zeros((_L,), jnp.int32)
  ones = jnp.ones((_L,), jnp.int32)

  def _zero(i, carry):
    flags_v[pl.ds(i * _L, _L)] = zeros
    return carry
  lax.fori_loop(0, _N_PAD // _L, _zero, 0, unroll=8)

  pltpu.sync_copy(edge_hbm.at[pl.ds(w * _E_PER_W, _E_PER_W)], idx_v)

  def _scatter(i, carry):
    ii = idx_v[pl.ds(i * _L, _L)]
    plsc.store_scatter(flags_v, [ii], ones)
    return carry
  lax.fori_loop(0, _E_PER_W // _L, _scatter, 0, unroll=8)

  def _pack(g, carry):
    a0 = flags_v[pl.ds(g * _G, _L)]
    a1 = flags_v[pl.ds(g * _G + _L, _L)]
    pb_v[pl.ds(g * _G, _G)] = plsc.pack(a0, a1, format=_PK)
    return carry
  lax.fori_loop(0, _N_PAD // _G, _pack, 0, unroll=4)

  pltpu.sync_copy(pb_v, pb_hbm.at[w])


_kernel_a = functools.partial(
    pl.kernel,
    out_type=jax.ShapeDtypeStruct((_NW, _N_PAD), jnp.int16),
    mesh=_MESH,
    compiler_params=_PARAMS,
    scratch_types=[
        pltpu.VMEM((_E_PER_W,), jnp.int32),
        pltpu.VMEM((_N_PAD,), jnp.int32),
        pltpu.VMEM((_N_PAD,), jnp.int16),
        pltpu.SemaphoreType.DMA,
    ],
)(_a_body)


def _c_body(pb_hbm, proj_hbm, out_hbm, stage_v, comb_v, lcomp_v, lcomp0_v,
            nodes_v, rows_v, sem):
  w = _wid()
  iota = lax.iota(jnp.int32, _L)
  zeros = jnp.zeros((_L,), jnp.int32)

  half = _N_PAD // 2
  for h in range(2):
    pltpu.sync_copy(pb_hbm.at[:, pl.ds(h * half, half)], stage_v)

    def _or(k, carry):
      acc = stage_v[0, pl.ds(k * _G, _G)]
      for t in range(1, _NW):
        acc = acc | stage_v[t, pl.ds(k * _G, _G)]
      comb_v[pl.ds(h * half + k * _G, _G)] = acc
      return carry
    lax.fori_loop(0, half // _G, _or, 0, unroll=2)

  cs = []
  for r in range(_NR):
    def _acc(g, carry):
      word = plsc.bitcast(comb_v[pl.ds(r * _RR + g * _G, _G)], jnp.int32)
      return carry + word
    accw = lax.fori_loop(0, _RR // _G, _acc, zeros, unroll=4)
    s = jnp.sum(accw)
    cs.append((s & 0xFFFF) + (s >> 16))
  offs = []
  tot = jnp.int32(0)
  for r in range(_NR):
    offs.append(tot)
    tot = tot + cs[r]
  total = tot

  j_lo = jnp.int32(w * _L)
  j_hi = jnp.minimum(j_lo + _L - 1, jnp.maximum(total - 1, 0))
  j_lo_c = jnp.minimum(j_lo, jnp.maximum(total - 1, 0))
  t_lo = jnp.int32(0)
  t_hi = jnp.int32(0)
  t0 = jnp.int32(0)
  off_lo = jnp.int32(0)
  for r in range(_NR):
    t_lo = t_lo + (offs[r] <= j_lo_c).astype(jnp.int32)
    t_hi = t_hi + (offs[r] <= j_hi).astype(jnp.int32)
    t0 = t0 + (offs[r] <= 0).astype(jnp.int32)
  t_lo = t_lo - 1
  t_hi = jnp.maximum(t_hi - 1, t_lo)
  t0 = t0 - 1
  for r in range(_NR):
    off_lo = off_lo + jnp.where(r < t_lo, cs[r], 0)

  def _compact_ranges(r_start, r_end, out_ref):
    def _outer(r, carry):
      def _inner(g, c2):
        b = comb_v[pl.ds(r * _RR + g * _G, _G)]
        subs = plsc.unpack(b, format=_PK)
        base = r * _RR + g * _G
        for k, f in enumerate(subs):
          f = f.astype(jnp.int32)
          m = f > 0
          pos = c2 + plsc.cumsum(f) - f
          vals = iota + (base + k * _L)
          plsc.store_scatter(out_ref, [pos], vals, mask=m)
          c2 = c2 + jnp.sum(f)
        return c2
      return lax.fori_loop(0, _RR // _G, _inner, carry)
    return lax.fori_loop(r_start, r_end, _outer, jnp.int32(0))

  _compact_ranges(t_lo, t_hi + 1, lcomp_v)
  _compact_ranges(t0, t0 + 1, lcomp0_v)
  node0 = lcomp0_v[pl.ds(0, _L)][0]

  jv = iota + j_lo
  valid = jv < total
  lidx = jnp.where(valid, jv - off_lo, zeros)
  node = plsc.load_gather(lcomp_v, [lidx])
  nodes_v[...] = jnp.where(valid, node, jnp.full((_L,), node0, jnp.int32))

  pltpu.async_copy(proj_hbm.at[nodes_v], rows_v, sem).wait()
  pltpu.sync_copy(rows_v, out_hbm.at[pl.ds(w * _L, _L)])


_kernel_c = functools.partial(
    pl.kernel,
    out_type=jax.ShapeDtypeStruct((_SEQ_P, _D), jnp.float32),
    mesh=_MESH,
    compiler_params=_PARAMS,
    scratch_types=[
        pltpu.VMEM((_NW, _N_PAD // 2), jnp.int16),
        pltpu.VMEM((_N_PAD,), jnp.int16),
        pltpu.VMEM((2048,), jnp.int32),
        pltpu.VMEM((_RR,), jnp.int32),
        pltpu.VMEM((_L,), jnp.int32),
        pltpu.VMEM((_L, _D), jnp.float32),
        pltpu.SemaphoreType.DMA,
    ],
)(_c_body)


def _mm_body(x_ref, wt_ref, b_ref, o_ref):
  o_ref[...] = jnp.dot(x_ref[...], wt_ref[...],
                       preferred_element_type=jnp.float32) + b_ref[...]


_project = pl.pallas_call(
    _mm_body,
    out_shape=jax.ShapeDtypeStruct((_N, _D), jnp.float32),
)


@jax.jit
def kernel(x, edge_index, W, b):
  proj = _project(x, W.T, b.reshape(1, _D))
  pbits = _kernel_a(edge_index[0])
  out = _kernel_c(pbits, proj)
  return out[None, :_SEQ, :]


# speedup vs baseline: 1.3440x; 1.3440x over previous
import functools

import jax
import jax.numpy as jnp
from jax import lax
from jax.experimental import pallas as pl
from jax.experimental.pallas import tpu as pltpu
from jax.experimental.pallas import tpu_sc as plsc

_N = 10000
_N_PAD = 10240
_E = 320000
_E_PER_W = _E // 32
_SEQ = 500
_SEQ_P = 512
_D = 128
_NW = 32
_L = 16
_NR = 20
_RR = 512

_MESH = plsc.VectorSubcoreMesh(core_axis_name="c", subcore_axis_name="s")
_PARAMS = pltpu.CompilerParams(needs_layout_passes=False,
                               use_tc_tiling_on_sc=False)


def _wid():
  return lax.axis_index("c") * 16 + lax.axis_index("s")


def _a_body(edge_hbm, zero_hbm, flags_hbm, idx_v, flags_v, sem):
  w = _wid()
  ones = jnp.ones((_L,), jnp.int32)

  pltpu.sync_copy(zero_hbm, flags_v)
  pltpu.sync_copy(edge_hbm.at[pl.ds(w * _E_PER_W, _E_PER_W)], idx_v)

  def _scatter(i, carry):
    ii = idx_v[pl.ds(i * _L, _L)]
    plsc.store_scatter(flags_v, [ii], ones)
    return carry
  lax.fori_loop(0, _E_PER_W // _L, _scatter, 0, unroll=16)

  pltpu.sync_copy(flags_v, flags_hbm.at[w])


_kernel_a = functools.partial(
    pl.kernel,
    out_type=jax.ShapeDtypeStruct((_NW, _N_PAD), jnp.int32),
    mesh=_MESH,
    compiler_params=_PARAMS,
    scratch_types=[
        pltpu.VMEM((_E_PER_W,), jnp.int32),
        pltpu.VMEM((_N_PAD,), jnp.int32),
        pltpu.SemaphoreType.DMA,
    ],
)(_a_body)


def _or_body(f_ref, o_ref):
  acc = f_ref[0]
  for t in range(1, _NW):
    acc = acc | f_ref[t]
  o_ref[...] = acc


_combine = pl.pallas_call(
    _or_body,
    out_shape=jax.ShapeDtypeStruct((_N_PAD // _D, _D), jnp.int32),
)


def _c_body(comb_hbm, proj_hbm, out_hbm, comb_v, lcomp_v, lcomp0_v,
            nodes_v, rows_v, sem):
  w = _wid()
  iota = lax.iota(jnp.int32, _L)
  zeros = jnp.zeros((_L,), jnp.int32)

  pltpu.sync_copy(comb_hbm, comb_v)

  cs = []
  for r in range(_NR):
    def _acc(g, carry):
      return carry + comb_v[pl.ds(r * _RR + g * _L, _L)]
    acc = lax.fori_loop(0, _RR // _L, _acc, zeros, unroll=4)
    cs.append(jnp.sum(acc))
  offs = []
  tot = jnp.int32(0)
  for r in range(_NR):
    offs.append(tot)
    tot = tot + cs[r]
  total = tot

  j_lo = jnp.int32(w * _L)
  j_hi = jnp.minimum(j_lo + _L - 1, jnp.maximum(total - 1, 0))
  j_lo_c = jnp.minimum(j_lo, jnp.maximum(total - 1, 0))
  t_lo = jnp.int32(0)
  t_hi = jnp.int32(0)
  t0 = jnp.int32(0)
  off_lo = jnp.int32(0)
  for r in range(_NR):
    t_lo = t_lo + (offs[r] <= j_lo_c).astype(jnp.int32)
    t_hi = t_hi + (offs[r] <= j_hi).astype(jnp.int32)
    t0 = t0 + (offs[r] <= 0).astype(jnp.int32)
  t_lo = t_lo - 1
  t_hi = jnp.maximum(t_hi - 1, t_lo)
  t0 = t0 - 1
  for r in range(_NR):
    off_lo = off_lo + jnp.where(r < t_lo, cs[r], 0)

  def _compact_ranges(r_start, r_end, out_ref):
    def _outer(r, carry):
      def _inner(g, c2):
        f = comb_v[pl.ds(r * _RR + g * _L, _L)]
        m = f > 0
        pos = c2 + plsc.cumsum(f) - f
        vals = iota + (r * _RR + g * _L)
        plsc.store_scatter(out_ref, [pos], vals, mask=m)
        return c2 + jnp.sum(f)
      return lax.fori_loop(0, _RR // _L, _inner, carry)
    return lax.fori_loop(r_start, r_end, _outer, jnp.int32(0))

  _compact_ranges(t_lo, t_hi + 1, lcomp_v)
  _compact_ranges(t0, t0 + 1, lcomp0_v)
  node0 = lcomp0_v[pl.ds(0, _L)][0]

  jv = iota + j_lo
  valid = jv < total
  lidx = jnp.where(valid, jv - off_lo, zeros)
  node = plsc.load_gather(lcomp_v, [lidx])
  nodes_v[...] = jnp.where(valid, node, jnp.full((_L,), node0, jnp.int32))

  pltpu.async_copy(proj_hbm.at[nodes_v], rows_v, sem).wait()
  pltpu.sync_copy(rows_v, out_hbm.at[pl.ds(w * _L, _L)])


_kernel_c = functools.partial(
    pl.kernel,
    out_type=jax.ShapeDtypeStruct((_SEQ_P, _D), jnp.float32),
    mesh=_MESH,
    compiler_params=_PARAMS,
    scratch_types=[
        pltpu.VMEM((_N_PAD,), jnp.int32),
        pltpu.VMEM((2048,), jnp.int32),
        pltpu.VMEM((_RR,), jnp.int32),
        pltpu.VMEM((_L,), jnp.int32),
        pltpu.VMEM((_L, _D), jnp.float32),
        pltpu.SemaphoreType.DMA,
    ],
)(_c_body)


def _mm_body(x_ref, wt_ref, b_ref, o_ref):
  o_ref[...] = jnp.dot(x_ref[...], wt_ref[...],
                       preferred_element_type=jnp.float32) + b_ref[...]


_project = pl.pallas_call(
    _mm_body,
    out_shape=jax.ShapeDtypeStruct((_N, _D), jnp.float32),
)


@jax.jit
def kernel(x, edge_index, W, b):
  proj = _project(x, W.T, b.reshape(1, _D))
  zero = jnp.zeros((_N_PAD,), jnp.int32)
  flags = _kernel_a(edge_index[0], zero)
  comb = _combine(flags.reshape(_NW, _N_PAD // _D, _D))
  out = _kernel_c(comb.reshape(_N_PAD), proj)
  return out[None, :_SEQ, :]


# speedup vs baseline: 1.4174x; 1.0546x over previous
import functools

import jax
import jax.numpy as jnp
from jax import lax
from jax.experimental import pallas as pl
from jax.experimental.pallas import tpu as pltpu
from jax.experimental.pallas import tpu_sc as plsc

_N = 10000
_N_PAD = 10240
_E = 320000
_E_PER_W = _E // 32
_SEQ = 500
_SEQ_P = 512
_D = 128
_NW = 32
_L = 16
_NR = 20
_RR = 512

_MESH = plsc.VectorSubcoreMesh(core_axis_name="c", subcore_axis_name="s")
_PARAMS = pltpu.CompilerParams(needs_layout_passes=False,
                               use_tc_tiling_on_sc=False)


def _wid():
  return lax.axis_index("c") * 16 + lax.axis_index("s")


def _a_body(edge_hbm, flags_hbm, idx_v, flags_v, sem):
  w = _wid()
  zeros = jnp.zeros((_L,), jnp.int32)
  ones = jnp.ones((_L,), jnp.int32)

  def _zero(i, carry):
    flags_v[pl.ds(i * _L, _L)] = zeros
    return carry
  lax.fori_loop(0, _N_PAD // _L, _zero, 0, unroll=8)

  pltpu.sync_copy(edge_hbm.at[pl.ds(w * _E_PER_W, _E_PER_W)], idx_v)

  def _scatter(i, carry):
    ii = idx_v[pl.ds(i * _L, _L)]
    plsc.store_scatter(flags_v, [ii], ones)
    return carry
  lax.fori_loop(0, _E_PER_W // _L, _scatter, 0, unroll=8)

  pltpu.sync_copy(flags_v, flags_hbm.at[w])


_kernel_a = functools.partial(
    pl.kernel,
    out_type=jax.ShapeDtypeStruct((_NW, _N_PAD), jnp.int32),
    mesh=_MESH,
    compiler_params=_PARAMS,
    scratch_types=[
        pltpu.VMEM((_E_PER_W,), jnp.int32),
        pltpu.VMEM((_N_PAD,), jnp.int32),
        pltpu.SemaphoreType.DMA,
    ],
)(_a_body)


def _or_body(f_ref, o_ref):
  acc = f_ref[0]
  for t in range(1, _NW):
    acc = acc | f_ref[t]
  o_ref[...] = acc


_combine = pl.pallas_call(
    _or_body,
    out_shape=jax.ShapeDtypeStruct((_N_PAD // _D, _D), jnp.int32),
)


def _c_body(comb_hbm, proj_hbm, out_hbm, comb_v, lcomp_v, lcomp0_v,
            nodes_v, rows_v, sem):
  w = _wid()
  iota = lax.iota(jnp.int32, _L)
  zeros = jnp.zeros((_L,), jnp.int32)

  pltpu.sync_copy(comb_hbm, comb_v)

  cs = []
  for r in range(_NR):
    def _acc(g, carry):
      return carry + comb_v[pl.ds(r * _RR + g * _L, _L)]
    acc = lax.fori_loop(0, _RR // _L, _acc, zeros, unroll=4)
    cs.append(jnp.sum(acc))
  offs = []
  tot = jnp.int32(0)
  for r in range(_NR):
    offs.append(tot)
    tot = tot + cs[r]
  total = tot

  j_lo = jnp.int32(w * _L)
  j_hi = jnp.minimum(j_lo + _L - 1, jnp.maximum(total - 1, 0))
  j_lo_c = jnp.minimum(j_lo, jnp.maximum(total - 1, 0))
  t_lo = jnp.int32(0)
  t_hi = jnp.int32(0)
  t0 = jnp.int32(0)
  off_lo = jnp.int32(0)
  for r in range(_NR):
    t_lo = t_lo + (offs[r] <= j_lo_c).astype(jnp.int32)
    t_hi = t_hi + (offs[r] <= j_hi).astype(jnp.int32)
    t0 = t0 + (offs[r] <= 0).astype(jnp.int32)
  t_lo = t_lo - 1
  t_hi = jnp.maximum(t_hi - 1, t_lo)
  t0 = t0 - 1
  for r in range(_NR):
    off_lo = off_lo + jnp.where(r < t_lo, cs[r], 0)

  def _compact_ranges(r_start, r_end, out_ref):
    def _outer(r, carry):
      def _inner(g, c2):
        f = comb_v[pl.ds(r * _RR + g * _L, _L)]
        m = f > 0
        pos = c2 + plsc.cumsum(f) - f
        vals = iota + (r * _RR + g * _L)
        plsc.store_scatter(out_ref, [pos], vals, mask=m)
        return c2 + jnp.sum(f)
      return lax.fori_loop(0, _RR // _L, _inner, carry)
    return lax.fori_loop(r_start, r_end, _outer, jnp.int32(0))

  _compact_ranges(t_lo, t_hi + 1, lcomp_v)
  _compact_ranges(t0, t0 + 1, lcomp0_v)
  node0 = lcomp0_v[pl.ds(0, _L)][0]

  jv = iota + j_lo
  valid = jv < total
  lidx = jnp.where(valid, jv - off_lo, zeros)
  node = plsc.load_gather(lcomp_v, [lidx])
  nodes_v[...] = jnp.where(valid, node, jnp.full((_L,), node0, jnp.int32))

  pltpu.async_copy(proj_hbm.at[nodes_v], rows_v, sem).wait()
  pltpu.sync_copy(rows_v, out_hbm.at[pl.ds(w * _L, _L)])


_kernel_c = functools.partial(
    pl.kernel,
    out_type=jax.ShapeDtypeStruct((_SEQ_P, _D), jnp.float32),
    mesh=_MESH,
    compiler_params=_PARAMS,
    scratch_types=[
        pltpu.VMEM((_N_PAD,), jnp.int32),
        pltpu.VMEM((2048,), jnp.int32),
        pltpu.VMEM((_RR,), jnp.int32),
        pltpu.VMEM((_L,), jnp.int32),
        pltpu.VMEM((_L, _D), jnp.float32),
        pltpu.SemaphoreType.DMA,
    ],
)(_c_body)


def _mm_body(x_ref, wt_ref, b_ref, o_ref):
  o_ref[...] = jnp.dot(x_ref[...], wt_ref[...],
                       preferred_element_type=jnp.float32) + b_ref[...]


_project = pl.pallas_call(
    _mm_body,
    out_shape=jax.ShapeDtypeStruct((_N, _D), jnp.float32),
)


@jax.jit
def kernel(x, edge_index, W, b):
  proj = _project(x, W.T, b.reshape(1, _D))
  flags = _kernel_a(edge_index[0])
  comb = _combine(flags.reshape(_NW, _N_PAD // _D, _D))
  out = _kernel_c(comb.reshape(_N_PAD), proj)
  return out[None, :_SEQ, :]
